# baseline (speedup 1.0000x reference)
import functools

import jax
import jax.numpy as jnp
from jax import lax
from jax.experimental import pallas as pl
from jax.experimental.pallas import tpu as pltpu
from jax.experimental.pallas import tpu_sc as plsc

N = 65536
E = 8
D_EMB = 32
D_SHS = 48
D_IN = 92
D_PAD = 128
D_H = 256
D_OUT = 59
D_OPAD = 128

MB = 512
NPAD = N + E * MB
NBLK = NPAD // MB

NW = 32
CH = 512
IW = 128
GPW = N // NW // CH

_ATTRS = ((3, 0), (3, 3), (4, 6), (1, 10), (48, 11), (32, 59), (1, 91))
_OUTS = ((3, 0), (3, 3), (4, 6), (1, 10), (48, 11))


def _sc_mesh():
    return plsc.VectorSubcoreMesh(core_axis_name="c", subcore_axis_name="s",
                                  num_cores=2, num_subcores=16)


@functools.cache
def _make_scatter_feat():
    @functools.partial(
        pl.kernel, mesh=_sc_mesh(),
        out_type=jax.ShapeDtypeStruct((NPAD, D_PAD), jnp.float32),
        scratch_types=[
            pltpu.VMEM((CH // IW, IW), jnp.int32),
            pltpu.VMEM((CH, D_PAD), jnp.float32),
            pltpu.VMEM((6160,), jnp.float32),
            pltpu.VMEM((CH * D_SHS,), jnp.float32),
            pltpu.VMEM((CH * D_EMB,), jnp.float32),
            pltpu.SemaphoreType.DMA,
        ],
        compiler_params=pltpu.CompilerParams(needs_layout_passes=False),
    )
    def scatter_feat(m_hbm, s_hbm, r_hbm, o_hbm, shs_hbm, emb_hbm, t_hbm,
                     dest3_hbm, out_hbm,
                     idx_v, featbuf, smallbuf, shsbuf, embbuf, sem):
        wid = lax.axis_index("s") * 2 + lax.axis_index("c")
        iota = lax.iota(jnp.int32, 16)
        zero = jnp.zeros((16,), jnp.float32)
        A = jnp.where(iota < 3, iota,
            jnp.where(iota < 6, 1536 + (iota - 3),
            jnp.where(iota < 10, 3072 + (iota - 6),
            jnp.where(iota < 11, 5120,
            jnp.where(iota < 12, 5632, 6144 + (iota - 12))))))
        B = jnp.where(iota < 6, 3,
            jnp.where(iota < 10, 4,
            jnp.where(iota < 12, 1, 0)))

        plsc.store_scatter(smallbuf, [6144 + iota], zero)

        def zbody(r, carry):
            featbuf[r, pl.ds(96, 16)] = zero
            featbuf[r, pl.ds(112, 16)] = zero
            return carry

        lax.fori_loop(0, CH, zbody, 0)

        for g in range(GPW):
            pbase = (wid * GPW + g) * CH
            pltpu.sync_copy(m_hbm.at[pl.ds(pbase * 3, CH * 3)],
                            smallbuf.at[pl.ds(0, CH * 3)])
            pltpu.sync_copy(s_hbm.at[pl.ds(pbase * 3, CH * 3)],
                            smallbuf.at[pl.ds(1536, CH * 3)])
            pltpu.sync_copy(r_hbm.at[pl.ds(pbase * 4, CH * 4)],
                            smallbuf.at[pl.ds(3072, CH * 4)])
            pltpu.sync_copy(o_hbm.at[pl.ds(pbase, CH)],
                            smallbuf.at[pl.ds(5120, CH)])
            pltpu.sync_copy(t_hbm.at[pl.ds(pbase, CH)],
                            smallbuf.at[pl.ds(5632, CH)])
            pltpu.sync_copy(shs_hbm.at[pl.ds(pbase * D_SHS, CH * D_SHS)],
                            shsbuf)
            pltpu.sync_copy(emb_hbm.at[pl.ds(pbase * D_EMB, CH * D_EMB)],
                            embbuf)
            pltpu.sync_copy(
                dest3_hbm.at[pl.ds((wid * GPW + g) * (CH // IW), CH // IW)],
                idx_v)

            def abody(p, carry):
                featbuf[p, pl.ds(0, 16)] = plsc.load_gather(
                    smallbuf, [A + p * B])
                for j in range(3):
                    featbuf[p, pl.ds(16 + j * 16, 16)] = plsc.load_gather(
                        shsbuf, [p * D_SHS + j * 16 + iota])
                for j in range(2):
                    featbuf[p, pl.ds(64 + j * 16, 16)] = plsc.load_gather(
                        embbuf, [p * D_EMB + j * 16 + iota])
                return carry

            lax.fori_loop(0, CH, abody, 0)

            handles = []
            for j in range(CH // IW):
                handles.append(pltpu.async_copy(
                    featbuf.at[pl.ds(j * IW, IW)],
                    out_hbm.at[idx_v.at[j]], sem))
            for h in handles:
                h.wait()
    return scatter_feat


def _scatter_feat(m, s, r, o, shs, emb, t, dest3):
    return _make_scatter_feat()(m, s, r, o, shs, emb, t, dest3)


@functools.cache
def _make_gather_combine():
    out_t = [
        jax.ShapeDtypeStruct((N * 3,), jnp.float32),
        jax.ShapeDtypeStruct((N * 3,), jnp.float32),
        jax.ShapeDtypeStruct((N * 4,), jnp.float32),
        jax.ShapeDtypeStruct((N,), jnp.float32),
        jax.ShapeDtypeStruct((N * D_SHS,), jnp.float32),
        jax.ShapeDtypeStruct((N,), jnp.float32),
    ]

    @functools.partial(
        pl.kernel, mesh=_sc_mesh(),
        out_type=out_t,
        scratch_types=[
            pltpu.VMEM((CH // IW, IW), jnp.int32),
            pltpu.VMEM((CH, D_OPAD), jnp.float32),
            pltpu.VMEM((5648,), jnp.float32),
            pltpu.VMEM((CH * D_SHS,), jnp.float32),
            pltpu.VMEM((CH,), jnp.float32),
            pltpu.VMEM((CH,), jnp.float32),
            pltpu.VMEM((CH,), jnp.float32),
            pltpu.VMEM((16,), jnp.float32),
            pltpu.SemaphoreType.DMA,
        ],
        compiler_params=pltpu.CompilerParams(needs_layout_passes=False),
    )
    def gather_combine(dsort_hbm, dest3_hbm, m_hbm, s_hbm, r_hbm, o_hbm,
                       shs_hbm, t0_hbm, t1_hbm, time_hbm,
                       mo_hbm, so_hbm, ro_hbm, oo_hbm, shso_hbm, mk_hbm,
                       idx_v, dbuf, smallbuf, shsbuf,
                       t0buf, t1buf, mkbuf, tsbuf, sem):
        wid = lax.axis_index("s") * 2 + lax.axis_index("c")
        iota = lax.iota(jnp.int32, 16)
        pltpu.sync_copy(time_hbm.at[pl.ds(0, 16)], tsbuf)
        ts = tsbuf[...][0]
        A2 = jnp.where(iota < 3, iota,
             jnp.where(iota < 6, 1536 + (iota - 3),
             jnp.where(iota < 10, 3072 + (iota - 6),
             jnp.where(iota < 11, 5120, 5632 + (iota - 11)))))
        B2 = jnp.where(iota < 6, 3,
             jnp.where(iota < 10, 4,
             jnp.where(iota < 11, 1, 0)))
        lane_op = (iota == 10)

        for g in range(GPW):
            pbase = (wid * GPW + g) * CH
            pltpu.sync_copy(
                dest3_hbm.at[pl.ds((wid * GPW + g) * (CH // IW), CH // IW)],
                idx_v)
            handles = []
            for j in range(CH // IW):
                handles.append(pltpu.async_copy(
                    dsort_hbm.at[idx_v.at[j]],
                    dbuf.at[pl.ds(j * IW, IW)], sem))
            pltpu.sync_copy(m_hbm.at[pl.ds(pbase * 3, CH * 3)],
                            smallbuf.at[pl.ds(0, CH * 3)])
            pltpu.sync_copy(s_hbm.at[pl.ds(pbase * 3, CH * 3)],
                            smallbuf.at[pl.ds(1536, CH * 3)])
            pltpu.sync_copy(r_hbm.at[pl.ds(pbase * 4, CH * 4)],
                            smallbuf.at[pl.ds(3072, CH * 4)])
            pltpu.sync_copy(o_hbm.at[pl.ds(pbase, CH)],
                            smallbuf.at[pl.ds(5120, CH)])
            pltpu.sync_copy(shs_hbm.at[pl.ds(pbase * D_SHS, CH * D_SHS)],
                            shsbuf)
            pltpu.sync_copy(t0_hbm.at[pl.ds(pbase, CH)], t0buf)
            pltpu.sync_copy(t1_hbm.at[pl.ds(pbase, CH)], t1buf)
            for h in handles:
                h.wait()

            def mbody(k, carry):
                prow = k * 16 + iota
                t0 = plsc.load_gather(t0buf, [prow])
                t1 = plsc.load_gather(t1buf, [prow])
                a = (ts >= t0) & (ts < t1)
                plsc.store_scatter(mkbuf, [prow], a.astype(jnp.float32))
                return carry

            lax.fori_loop(0, CH // 16, mbody, 0)

            def cbody(p, carry):
                av = plsc.load_gather(mkbuf, [iota * 0 + p])
                a = av > 0.0
                ia = A2 + p * B2
                u = plsc.load_gather(smallbuf, [ia])
                d = dbuf[p, pl.ds(0, 16)]
                res = jnp.where(a, u + d, jnp.where(lane_op, -100.0, u))
                plsc.store_scatter(smallbuf, [ia], res)
                for j in range(3):
                    isj = p * D_SHS + j * 16 + iota
                    us = plsc.load_gather(shsbuf, [isj])
                    ds_ = dbuf[p, pl.ds(11 + j * 16, 16)]
                    plsc.store_scatter(shsbuf, [isj],
                                       jnp.where(a, us + ds_, us))
                return carry

            lax.fori_loop(0, CH, cbody, 0)

            pltpu.sync_copy(smallbuf.at[pl.ds(0, CH * 3)],
                            mo_hbm.at[pl.ds(pbase * 3, CH * 3)])
            pltpu.sync_copy(smallbuf.at[pl.ds(1536, CH * 3)],
                            so_hbm.at[pl.ds(pbase * 3, CH * 3)])
            pltpu.sync_copy(smallbuf.at[pl.ds(3072, CH * 4)],
                            ro_hbm.at[pl.ds(pbase * 4, CH * 4)])
            pltpu.sync_copy(smallbuf.at[pl.ds(5120, CH)],
                            oo_hbm.at[pl.ds(pbase, CH)])
            pltpu.sync_copy(shsbuf, shso_hbm.at[pl.ds(pbase * D_SHS,
                                                      CH * D_SHS)])
            pltpu.sync_copy(mkbuf, mk_hbm.at[pl.ds(pbase, CH)])
    return gather_combine


def _gather_combine(dsort, dest3, m, s, r, o, shs, t0, t1, time_flat):
    return _make_gather_combine()(dsort, dest3, m, s, r, o, shs, t0, t1,
                                  time_flat)


def _mm_body(eid_ref, x_ref, W1_ref, b1_ref, W2_ref, b2_ref, o_ref):
    x = x_ref[...]
    h = jnp.tanh(jnp.dot(x, W1_ref[0], preferred_element_type=jnp.float32)
                 + b1_ref[0])
    o_ref[...] = (jnp.dot(h, W2_ref[0], preferred_element_type=jnp.float32)
                  + b2_ref[0])


def _grouped_mm(block_eid, feat_sorted, W1p, b1, W2p, b2p):
    grid_spec = pltpu.PrefetchScalarGridSpec(
        num_scalar_prefetch=1,
        grid=(NBLK,),
        in_specs=[
            pl.BlockSpec((MB, D_PAD), lambda i, eid: (i, 0)),
            pl.BlockSpec((1, D_PAD, D_H), lambda i, eid: (eid[i], 0, 0)),
            pl.BlockSpec((1, 1, D_H), lambda i, eid: (eid[i], 0, 0)),
            pl.BlockSpec((1, D_H, D_OPAD), lambda i, eid: (eid[i], 0, 0)),
            pl.BlockSpec((1, 1, D_OPAD), lambda i, eid: (eid[i], 0, 0)),
        ],
        out_specs=pl.BlockSpec((MB, D_OPAD), lambda i, eid: (i, 0)),
    )
    return pl.pallas_call(
        _mm_body,
        grid_spec=grid_spec,
        out_shape=jax.ShapeDtypeStruct((NPAD, D_OPAD), jnp.float32),
        compiler_params=pltpu.CompilerParams(
            dimension_semantics=("arbitrary",)),
    )(block_eid, feat_sorted, W1p, b1.reshape(E, 1, D_H), W2p,
      b2p.reshape(E, 1, D_OPAD))


def kernel(means3D, scales, rotations, opacity, shs, time, embeddings,
           seg_id_g, t_start_g, t_end_g, W1, b1, W2, b2):
    n = means3D.shape[0]
    seg = seg_id_g.astype(jnp.int32)

    onehot = (seg[:, None] == jnp.arange(E, dtype=jnp.int32)[None, :])
    counts = jnp.sum(onehot.astype(jnp.int32), axis=0)
    rank = (jnp.take_along_axis(jnp.cumsum(onehot.astype(jnp.int32), axis=0),
                                seg[:, None], axis=1)[:, 0] - 1)
    padded = ((counts + MB - 1) // MB) * MB
    seg_base = jnp.concatenate(
        [jnp.zeros((1,), jnp.int32), jnp.cumsum(padded)[:-1]])
    dest = seg_base[seg] + rank
    dest3 = dest.reshape(N // IW, IW)
    block_start = jnp.arange(NBLK, dtype=jnp.int32) * MB
    block_eid = jnp.clip(
        jnp.sum(block_start[:, None] >= seg_base[None, :], axis=1) - 1,
        0, E - 1).astype(jnp.int32)

    W1p = jnp.concatenate([
        W1[:, 0:11, :], W1[:, 91:92, :],
        jnp.zeros((E, 4, D_H), jnp.float32),
        W1[:, 11:59, :], W1[:, 59:91, :],
        jnp.zeros((E, D_PAD - 96, D_H), jnp.float32)], axis=1)
    W2p = jnp.pad(W2, ((0, 0), (0, 0), (0, D_OPAD - D_OUT)))
    b2p = jnp.pad(b2, ((0, 0), (0, D_OPAD - D_OUT)))

    m_fl = means3D.reshape(-1)
    s_fl = scales.reshape(-1)
    r_fl = rotations.reshape(-1)
    o_fl = opacity.reshape(-1)
    shs_fl = shs.reshape(-1)
    emb_fl = embeddings.reshape(-1)
    t_fl = time.reshape(-1)

    feat_sorted = _scatter_feat(m_fl, s_fl, r_fl, o_fl, shs_fl, emb_fl,
                                t_fl, dest3)
    delta_sorted = _grouped_mm(block_eid, feat_sorted, W1p, b1, W2p, b2p)
    mo, so, ro, oo, shso, mko = _gather_combine(
        delta_sorted, dest3, m_fl, s_fl, r_fl, o_fl, shs_fl,
        t_start_g, t_end_g, t_fl)

    return (mo.reshape(n, 3), so.reshape(n, 3), ro.reshape(n, 4),
            oo.reshape(n, 1), shso.reshape(n, 16, 3),
            mko.astype(bool))


# speedup vs baseline: 2.9690x; 2.9690x over previous
import jax
import jax.numpy as jnp
from jax.experimental import pallas as pl
from jax.experimental.pallas import tpu as pltpu

N = 65536
E = 8
D_SHS = 48
D_IN = 92
D_H = 256
D_OUT = 59

_BLK = 2048


def _body(ts_ref, x_ref, W1_ref, b1_ref, W2_ref, b2_ref, out_ref):
    ts = ts_ref[0, 0]
    x = x_ref[...]
    feat = x[:, 0:D_IN]
    tstart = x[:, D_IN:D_IN + 1]
    tend = x[:, D_IN + 1:D_IN + 2]
    seg = x[:, D_IN + 2:D_IN + 3]
    delta = jnp.zeros((x.shape[0], D_OUT), jnp.float32)
    for e in range(E):
        h = jnp.tanh(
            jnp.dot(feat, W1_ref[e], preferred_element_type=jnp.float32)
            + b1_ref[e:e + 1, :])
        d = (jnp.dot(h, W2_ref[e], preferred_element_type=jnp.float32)
             + b2_ref[e:e + 1, :])
        delta = delta + jnp.where(seg == float(e), d, 0.0)
    active = (ts >= tstart) & (ts < tend)
    m = x[:, 0:3]
    s = x[:, 3:6]
    r = x[:, 6:10]
    o = x[:, 10:11]
    shs = x[:, 11:59]
    out = jnp.concatenate([
        jnp.where(active, m + delta[:, 0:3], m),
        jnp.where(active, s + delta[:, 3:6], s),
        jnp.where(active, r + delta[:, 6:10], r),
        jnp.where(active, o + delta[:, 10:11], -100.0),
        jnp.where(active, shs + delta[:, 11:59], shs),
        active.astype(jnp.float32),
        jnp.zeros((x.shape[0], 128 - D_OUT - 1), jnp.float32),
    ], axis=1)
    out_ref[...] = out


@jax.jit
def _run(ts, packed, W1, b1, W2, b2):
    nblk = N // _BLK
    row = lambda i: (i, 0)
    fixed2 = lambda i: (0, 0)
    fixed3 = lambda i: (0, 0, 0)
    return pl.pallas_call(
        _body,
        grid=(nblk,),
        in_specs=[
            pl.BlockSpec((1, 1), fixed2),
            pl.BlockSpec((_BLK, 128), row),
            pl.BlockSpec((E, D_IN, D_H), fixed3),
            pl.BlockSpec((E, D_H), fixed2),
            pl.BlockSpec((E, D_H, D_OUT), fixed3),
            pl.BlockSpec((E, D_OUT), fixed2),
        ],
        out_specs=pl.BlockSpec((_BLK, 128), row),
        out_shape=jax.ShapeDtypeStruct((N, 128), jnp.float32),
        compiler_params=pltpu.CompilerParams(
            dimension_semantics=("parallel",)),
    )(ts, packed, W1, b1, W2, b2)


def kernel(means3D, scales, rotations, opacity, shs, time, embeddings,
           seg_id_g, t_start_g, t_end_g, W1, b1, W2, b2):
    n = means3D.shape[0]
    shs2 = shs.reshape(n, D_SHS)
    packed = jnp.concatenate([
        means3D, scales, rotations, opacity, shs2, embeddings, time,
        t_start_g.reshape(n, 1), t_end_g.reshape(n, 1),
        seg_id_g.astype(jnp.float32).reshape(n, 1),
        jnp.zeros((n, 128 - D_IN - 3), jnp.float32),
    ], axis=1)
    ts = time.reshape(-1)[0].reshape(1, 1)
    out = _run(ts, packed, W1, b1, W2, b2)
    return (out[:, 0:3], out[:, 3:6], out[:, 6:10], out[:, 10:11],
            out[:, 11:59].reshape(n, 16, 3), out[:, 59] > 0.0)


# speedup vs baseline: 3.0208x; 1.0174x over previous
import functools

import jax
import jax.numpy as jnp
from jax import lax
from jax.experimental import pallas as pl
from jax.experimental.pallas import tpu as pltpu
from jax.experimental.pallas import tpu_sc as plsc

N = 65536
E = 8
D_EMB = 32
D_SHS = 48
D_IN = 92
D_PAD = 128
D_H = 256
D_OUT = 59
D_OPAD = 128

MB = 512
NPAD = N + E * MB
NBLK = NPAD // MB

NW = 32
CH = 512
IW = 128
GPW = N // NW // CH

def _sc_mesh():
    return plsc.VectorSubcoreMesh(core_axis_name="c", subcore_axis_name="s",
                                  num_cores=2, num_subcores=16)


@functools.cache
def _make_scatter_feat():
    @functools.partial(
        pl.kernel, mesh=_sc_mesh(),
        out_type=jax.ShapeDtypeStruct((NPAD, D_PAD), jnp.float32),
        scratch_types=[
            pltpu.VMEM((CH // IW, IW), jnp.int32),
            pltpu.VMEM((CH, D_PAD), jnp.float32),
            pltpu.SemaphoreType.DMA,
        ],
    )
    def scatter_feat(feat_hbm, dest3_hbm, out_hbm, idx_v, rows_v, sem):
        wid = lax.axis_index("s") * 2 + lax.axis_index("c")
        for g in range(GPW):
            base = (wid * GPW + g) * CH
            pltpu.sync_copy(feat_hbm.at[pl.ds(base, CH)], rows_v)
            pltpu.sync_copy(dest3_hbm.at[pl.ds(wid * GPW * (CH // IW)
                                               + g * (CH // IW), CH // IW)],
                            idx_v)
            handles = []
            for j in range(CH // IW):
                handles.append(pltpu.async_copy(
                    rows_v.at[pl.ds(j * IW, IW)],
                    out_hbm.at[idx_v.at[j]], sem))
            for h in handles:
                h.wait()
    return scatter_feat


def _scatter_feat(feat, dest3):
    return _make_scatter_feat()(feat, dest3)


@functools.cache
def _make_gather_delta():
    @functools.partial(
        pl.kernel, mesh=_sc_mesh(),
        out_type=jax.ShapeDtypeStruct((N, D_OPAD), jnp.float32),
        scratch_types=[
            pltpu.VMEM((CH // IW, IW), jnp.int32),
            pltpu.VMEM((CH, D_OPAD), jnp.float32),
            pltpu.SemaphoreType.DMA,
        ],
    )
    def gather_delta(dsort_hbm, dest3_hbm, out_hbm, idx_v, rows_v, sem):
        wid = lax.axis_index("s") * 2 + lax.axis_index("c")
        for g in range(GPW):
            base = (wid * GPW + g) * CH
            pltpu.sync_copy(dest3_hbm.at[pl.ds(wid * GPW * (CH // IW)
                                               + g * (CH // IW), CH // IW)],
                            idx_v)
            handles = []
            for j in range(CH // IW):
                handles.append(pltpu.async_copy(
                    dsort_hbm.at[idx_v.at[j]],
                    rows_v.at[pl.ds(j * IW, IW)], sem))
            for h in handles:
                h.wait()
            pltpu.sync_copy(rows_v, out_hbm.at[pl.ds(base, CH)])
    return gather_delta


def _gather_delta(dsort, dest3):
    return _make_gather_delta()(dsort, dest3)


def _mm_body(eid_ref, ts_ref, x_ref, W1_ref, b1_ref, W2_ref, b2_ref,
             o_ref):
    ts = ts_ref[0, 0]
    x = x_ref[...]
    h = jnp.tanh(jnp.dot(x, W1_ref[0], preferred_element_type=jnp.float32)
                 + b1_ref[0])
    d = (jnp.dot(h, W2_ref[0], preferred_element_type=jnp.float32)
         + b2_ref[0])
    m = x[:, 0:3]
    s = x[:, 3:6]
    r = x[:, 6:10]
    o = x[:, 10:11]
    shs = x[:, 11:59]
    t0 = x[:, 92:93]
    t1 = x[:, 93:94]
    active = (ts >= t0) & (ts < t1)
    o_ref[...] = jnp.concatenate([
        jnp.where(active, m + d[:, 0:3], m),
        jnp.where(active, s + d[:, 3:6], s),
        jnp.where(active, r + d[:, 6:10], r),
        jnp.where(active, o + d[:, 10:11], -100.0),
        jnp.where(active, shs + d[:, 11:59], shs),
        active.astype(jnp.float32),
        jnp.zeros((x.shape[0], D_OPAD - D_OUT - 1), jnp.float32),
    ], axis=1)


def _grouped_mm(block_eid, ts, feat_sorted, W1p, b1, W2p, b2p):
    grid_spec = pltpu.PrefetchScalarGridSpec(
        num_scalar_prefetch=1,
        grid=(NBLK,),
        in_specs=[
            pl.BlockSpec((1, 1), lambda i, eid: (0, 0)),
            pl.BlockSpec((MB, D_PAD), lambda i, eid: (i, 0)),
            pl.BlockSpec((1, D_PAD, D_H), lambda i, eid: (eid[i], 0, 0)),
            pl.BlockSpec((1, 1, D_H), lambda i, eid: (eid[i], 0, 0)),
            pl.BlockSpec((1, D_H, D_OPAD), lambda i, eid: (eid[i], 0, 0)),
            pl.BlockSpec((1, 1, D_OPAD), lambda i, eid: (eid[i], 0, 0)),
        ],
        out_specs=pl.BlockSpec((MB, D_OPAD), lambda i, eid: (i, 0)),
    )
    return pl.pallas_call(
        _mm_body,
        grid_spec=grid_spec,
        out_shape=jax.ShapeDtypeStruct((NPAD, D_OPAD), jnp.float32),
        compiler_params=pltpu.CompilerParams(
            dimension_semantics=("arbitrary",)),
    )(block_eid, ts, feat_sorted, W1p, b1.reshape(E, 1, D_H), W2p,
      b2p.reshape(E, 1, D_OPAD))


def _combine_body(ts_ref, m_ref, s_ref, r_ref, o_ref, shs_ref,
                  tstart_ref, tend_ref, d_ref,
                  m_out, s_out, r_out, o_out, shs_out, mask_out):
    ts = ts_ref[0, 0]
    m = m_ref[...]
    s = s_ref[...]
    r = r_ref[...]
    o = o_ref[...]
    shs = shs_ref[...]
    d = d_ref[...]
    active = (ts >= tstart_ref[...]) & (ts < tend_ref[...])
    m_out[...] = jnp.where(active, m + d[:, 0:3], m)
    s_out[...] = jnp.where(active, s + d[:, 3:6], s)
    r_out[...] = jnp.where(active, r + d[:, 6:10], r)
    o_out[...] = jnp.where(active, o + d[:, 10:11], -100.0)
    shs_out[...] = jnp.where(active, shs + d[:, 11:59], shs)
    mask_out[...] = active.astype(jnp.float32)


def _combine(ts, means3D, scales, rotations, opacity, shs2, tstart, tend,
             delta):
    B = 2048
    row = lambda i: (i, 0)
    fixed = lambda i: (0, 0)
    return pl.pallas_call(
        _combine_body,
        grid=(N // B,),
        in_specs=[
            pl.BlockSpec((1, 1), fixed),
            pl.BlockSpec((B, 3), row),
            pl.BlockSpec((B, 3), row),
            pl.BlockSpec((B, 4), row),
            pl.BlockSpec((B, 1), row),
            pl.BlockSpec((B, D_SHS), row),
            pl.BlockSpec((B, 1), row),
            pl.BlockSpec((B, 1), row),
            pl.BlockSpec((B, D_OPAD), row),
        ],
        out_specs=[
            pl.BlockSpec((B, 3), row),
            pl.BlockSpec((B, 3), row),
            pl.BlockSpec((B, 4), row),
            pl.BlockSpec((B, 1), row),
            pl.BlockSpec((B, D_SHS), row),
            pl.BlockSpec((B, 1), row),
        ],
        out_shape=[
            jax.ShapeDtypeStruct((N, 3), jnp.float32),
            jax.ShapeDtypeStruct((N, 3), jnp.float32),
            jax.ShapeDtypeStruct((N, 4), jnp.float32),
            jax.ShapeDtypeStruct((N, 1), jnp.float32),
            jax.ShapeDtypeStruct((N, D_SHS), jnp.float32),
            jax.ShapeDtypeStruct((N, 1), jnp.float32),
        ],
        compiler_params=pltpu.CompilerParams(
            dimension_semantics=("parallel",)),
    )(ts, means3D, scales, rotations, opacity, shs2, tstart, tend, delta)


def kernel(means3D, scales, rotations, opacity, shs, time, embeddings,
           seg_id_g, t_start_g, t_end_g, W1, b1, W2, b2):
    n = means3D.shape[0]
    shs2 = shs.reshape(n, D_SHS)
    seg = seg_id_g.astype(jnp.int32)
    tstart = t_start_g.reshape(n, 1)
    tend = t_end_g.reshape(n, 1)
    ts = time.reshape(-1)[0].reshape(1, 1)

    onehot = (seg[:, None] == jnp.arange(E, dtype=jnp.int32)[None, :])
    counts = jnp.sum(onehot.astype(jnp.int32), axis=0)
    rank = (jnp.take_along_axis(jnp.cumsum(onehot.astype(jnp.int32), axis=0),
                                seg[:, None], axis=1)[:, 0] - 1)
    padded = ((counts + MB - 1) // MB) * MB
    seg_base = jnp.concatenate(
        [jnp.zeros((1,), jnp.int32), jnp.cumsum(padded)[:-1]])
    dest = seg_base[seg] + rank
    dest3 = dest.reshape(N // IW, IW)
    block_start = jnp.arange(NBLK, dtype=jnp.int32) * MB
    block_eid = jnp.clip(
        jnp.sum(block_start[:, None] >= seg_base[None, :], axis=1) - 1,
        0, E - 1).astype(jnp.int32)

    feat = jnp.concatenate(
        [means3D, scales, rotations, opacity, shs2, embeddings, time,
         tstart, tend,
         jnp.zeros((n, D_PAD - D_IN - 2), jnp.float32)], axis=1)

    W1p = jnp.pad(W1, ((0, 0), (0, D_PAD - D_IN), (0, 0)))
    W2p = jnp.pad(W2, ((0, 0), (0, 0), (0, D_OPAD - D_OUT)))
    b2p = jnp.pad(b2, ((0, 0), (0, D_OPAD - D_OUT)))

    feat_sorted = _scatter_feat(feat, dest3)
    res_sorted = _grouped_mm(block_eid, ts, feat_sorted, W1p, b1, W2p, b2p)
    res = _gather_delta(res_sorted, dest3)

    return (res[:, 0:3], res[:, 3:6], res[:, 6:10], res[:, 10:11],
            res[:, 11:59].reshape(n, 16, 3), res[:, 59] > 0.0)


# speedup vs baseline: 3.0285x; 1.0026x over previous
import functools

import jax
import jax.numpy as jnp
from jax import lax
from jax.experimental import pallas as pl
from jax.experimental.pallas import tpu as pltpu
from jax.experimental.pallas import tpu_sc as plsc

N = 65536
E = 8
D_EMB = 32
D_SHS = 48
D_IN = 92
D_PAD = 128
D_H = 256
D_OUT = 59
D_OPAD = 128

MB = 512
NPAD = N + E * MB
NBLK = NPAD // MB

NW = 32
CH = 512
IW = 128
GPW = N // NW // CH

def _sc_mesh():
    return plsc.VectorSubcoreMesh(core_axis_name="c", subcore_axis_name="s",
                                  num_cores=2, num_subcores=16)


@functools.cache
def _make_scatter_feat():
    @functools.partial(
        pl.kernel, mesh=_sc_mesh(),
        out_type=jax.ShapeDtypeStruct((NPAD, D_PAD), jnp.float32),
        scratch_types=[
            pltpu.VMEM((CH // IW, IW), jnp.int32),
            pltpu.VMEM((CH, D_PAD), jnp.float32),
            pltpu.SemaphoreType.DMA,
        ],
    )
    def scatter_feat(feat_hbm, dest3_hbm, out_hbm, idx_v, rows_v, sem):
        wid = lax.axis_index("s") * 2 + lax.axis_index("c")
        for g in range(GPW):
            base = (wid * GPW + g) * CH
            pltpu.sync_copy(feat_hbm.at[pl.ds(base, CH)], rows_v)
            pltpu.sync_copy(dest3_hbm.at[pl.ds(wid * GPW * (CH // IW)
                                               + g * (CH // IW), CH // IW)],
                            idx_v)
            handles = []
            for j in range(CH // IW):
                handles.append(pltpu.async_copy(
                    rows_v.at[pl.ds(j * IW, IW)],
                    out_hbm.at[idx_v.at[j]], sem))
            for h in handles:
                h.wait()
    return scatter_feat


def _scatter_feat(feat, dest3):
    return _make_scatter_feat()(feat, dest3)


@functools.cache
def _make_gather_delta():
    @functools.partial(
        pl.kernel, mesh=_sc_mesh(),
        out_type=jax.ShapeDtypeStruct((N, D_OPAD), jnp.float32),
        scratch_types=[
            pltpu.VMEM((CH // IW, IW), jnp.int32),
            pltpu.VMEM((CH, D_OPAD), jnp.float32),
            pltpu.SemaphoreType.DMA,
        ],
    )
    def gather_delta(dsort_hbm, dest3_hbm, out_hbm, idx_v, rows_v, sem):
        wid = lax.axis_index("s") * 2 + lax.axis_index("c")
        for g in range(GPW):
            base = (wid * GPW + g) * CH
            pltpu.sync_copy(dest3_hbm.at[pl.ds(wid * GPW * (CH // IW)
                                               + g * (CH // IW), CH // IW)],
                            idx_v)
            handles = []
            for j in range(CH // IW):
                handles.append(pltpu.async_copy(
                    dsort_hbm.at[idx_v.at[j]],
                    rows_v.at[pl.ds(j * IW, IW)], sem))
            for h in handles:
                h.wait()
            pltpu.sync_copy(rows_v, out_hbm.at[pl.ds(base, CH)])
    return gather_delta


def _gather_delta(dsort, dest3):
    return _make_gather_delta()(dsort, dest3)


def _mm_body(eid_ref, ts_ref, x_ref, W1_ref, b1_ref, W2_ref, b2_ref,
             o_ref):
    ts = ts_ref[0, 0]
    x = x_ref[...]
    h = jnp.tanh(jnp.dot(x, W1_ref[0], preferred_element_type=jnp.float32)
                 + b1_ref[0])
    d = (jnp.dot(h, W2_ref[0], preferred_element_type=jnp.float32)
         + b2_ref[0])
    m = x[:, 0:3]
    s = x[:, 3:6]
    r = x[:, 6:10]
    o = x[:, 10:11]
    shs = x[:, 11:59]
    t0 = x[:, 92:93]
    t1 = x[:, 93:94]
    active = (ts >= t0) & (ts < t1)
    o_ref[...] = jnp.concatenate([
        jnp.where(active, m + d[:, 0:3], m),
        jnp.where(active, s + d[:, 3:6], s),
        jnp.where(active, r + d[:, 6:10], r),
        jnp.where(active, o + d[:, 10:11], -100.0),
        jnp.where(active, shs + d[:, 11:59], shs),
        active.astype(jnp.float32),
        jnp.zeros((x.shape[0], D_OPAD - D_OUT - 1), jnp.float32),
    ], axis=1)


def _grouped_mm(block_eid, ts, feat_sorted, W1p, b1, W2p, b2p):
    grid_spec = pltpu.PrefetchScalarGridSpec(
        num_scalar_prefetch=1,
        grid=(NBLK,),
        in_specs=[
            pl.BlockSpec((1, 1), lambda i, eid: (0, 0)),
            pl.BlockSpec((MB, D_PAD), lambda i, eid: (i, 0)),
            pl.BlockSpec((1, D_PAD, D_H), lambda i, eid: (eid[i], 0, 0)),
            pl.BlockSpec((1, 1, D_H), lambda i, eid: (eid[i], 0, 0)),
            pl.BlockSpec((1, D_H, D_OPAD), lambda i, eid: (eid[i], 0, 0)),
            pl.BlockSpec((1, 1, D_OPAD), lambda i, eid: (eid[i], 0, 0)),
        ],
        out_specs=pl.BlockSpec((MB, D_OPAD), lambda i, eid: (i, 0)),
    )
    return pl.pallas_call(
        _mm_body,
        grid_spec=grid_spec,
        out_shape=jax.ShapeDtypeStruct((NPAD, D_OPAD), jnp.float32),
        compiler_params=pltpu.CompilerParams(
            dimension_semantics=("arbitrary",)),
    )(block_eid, ts, feat_sorted, W1p, b1.reshape(E, 1, D_H), W2p,
      b2p.reshape(E, 1, D_OPAD))


def kernel(means3D, scales, rotations, opacity, shs, time, embeddings,
           seg_id_g, t_start_g, t_end_g, W1, b1, W2, b2):
    n = means3D.shape[0]
    shs2 = shs.reshape(n, D_SHS)
    seg = seg_id_g.astype(jnp.int32)
    tstart = t_start_g.reshape(n, 1)
    tend = t_end_g.reshape(n, 1)
    ts = time.reshape(-1)[0].reshape(1, 1)

    onehot = (seg[:, None] == jnp.arange(E, dtype=jnp.int32)[None, :])
    counts = jnp.sum(onehot.astype(jnp.int32), axis=0)
    rank = (jnp.take_along_axis(jnp.cumsum(onehot.astype(jnp.int32), axis=0),
                                seg[:, None], axis=1)[:, 0] - 1)
    padded = ((counts + MB - 1) // MB) * MB
    seg_base = jnp.concatenate(
        [jnp.zeros((1,), jnp.int32), jnp.cumsum(padded)[:-1]])
    dest = seg_base[seg] + rank
    dest3 = dest.reshape(N // IW, IW)
    block_start = jnp.arange(NBLK, dtype=jnp.int32) * MB
    block_eid = jnp.clip(
        jnp.sum(block_start[:, None] >= seg_base[None, :], axis=1) - 1,
        0, E - 1).astype(jnp.int32)

    feat = jnp.concatenate(
        [means3D, scales, rotations, opacity, shs2, embeddings, time,
         tstart, tend,
         jnp.zeros((n, D_PAD - D_IN - 2), jnp.float32)], axis=1)

    W1p = jnp.pad(W1, ((0, 0), (0, D_PAD - D_IN), (0, 0)))
    W2p = jnp.pad(W2, ((0, 0), (0, 0), (0, D_OPAD - D_OUT)))
    b2p = jnp.pad(b2, ((0, 0), (0, D_OPAD - D_OUT)))

    feat_sorted = _scatter_feat(feat, dest3)
    res_sorted = _grouped_mm(block_eid, ts, feat_sorted, W1p, b1, W2p, b2p)
    res = _gather_delta(res_sorted, dest3)

    return (res[:, 0:3], res[:, 3:6], res[:, 6:10], res[:, 10:11],
            res[:, 11:59].reshape(n, 16, 3), res[:, 59] > 0.0)
